# baseline (speedup 1.0000x reference)
import functools
import math

import jax
import jax.numpy as jnp
from jax import lax
from jax.experimental import pallas as pl
from jax.experimental.pallas import tpu as pltpu
from jax.experimental.pallas import tpu_sc as plsc

N = 4_000_000
R_BINS = 20
NUM_TILES = 32
PARTS = ((2_880_000, 5000, 18), (1_120_000, 5000, 7))
VECS = 320
UNROLL = 4
BUF = VECS * 16
HIST_ROWS = R_BINS * R_BINS + 1
CLAMP = float(R_BINS * R_BINS)

_mesh = plsc.VectorSubcoreMesh(core_axis_name="c", subcore_axis_name="s")


@functools.lru_cache(maxsize=None)
def _make_sc_hist(n, chunk, steps):
    per_tile = n // NUM_TILES
    assert chunk * steps == per_tile and chunk % 8 == 0 and per_tile % 8 == 0
    assert chunk <= VECS * 16

    @functools.partial(
        pl.kernel,
        out_type=jax.ShapeDtypeStruct((64,), jnp.float32),
        mesh=_mesh,
        compiler_params=pltpu.CompilerParams(needs_layout_passes=False),
        scratch_types=[
            pltpu.VMEM((BUF,), jnp.float32),
            pltpu.VMEM((BUF,), jnp.float32),
            pltpu.VMEM((BUF,), jnp.float32),
            pltpu.VMEM((BUF,), jnp.float32),
            pltpu.VMEM((BUF,), jnp.float32),
            pltpu.VMEM((BUF,), jnp.float32),
            pltpu.VMEM((HIST_ROWS * 16,), jnp.float32),
            pltpu.VMEM((R_BINS * 16,), jnp.float32),
            pltpu.VMEM((16 * R_BINS * 16,), jnp.float32),
            pltpu.VMEM((32,), jnp.float32),
            pltpu.VMEM_SHARED((16 * R_BINS * 16,), jnp.float32),
            pltpu.SemaphoreType.DMA,
            pltpu.SemaphoreType.DMA,
        ],
    )
    def _sc_hist(x_hbm, y_hbm, mass_hbm, out_hbm, x_a, x_b, y_a, y_b, m_a, m_b,
                 hist, bins, stage, outbuf, shared, sem_a, sem_b):
        cid = lax.axis_index("c")
        sid = lax.axis_index("s")
        wid = sid * 2 + cid
        pbase = wid * per_tile

        zero16 = jnp.zeros((16,), jnp.float32)
        lane = lax.iota(jnp.int32, 16)

        x_bufs = (x_a, x_b)
        y_bufs = (y_a, y_b)
        m_bufs = (m_a, m_b)
        sems = (sem_a, sem_b)

        if chunk % 16:
            pad = ((chunk + 15) // 16) * 16 - 16
            for buf in (x_a, x_b, y_a, y_b, m_a, m_b):
                buf[pl.ds(pad, 16)] = zero16

        def _zh(i, c):
            hist[pl.ds(i * 16, 16)] = zero16
            return c
        lax.fori_loop(0, HIST_ROWS, _zh, 0)

        def dma_in(g, slot):
            p0 = pbase + g * chunk
            cps = tuple(
                pltpu.make_async_copy(
                    hbm.at[pl.ds(p0, chunk)],
                    buf[slot].at[pl.ds(0, chunk)], sems[slot])
                for hbm, buf in ((x_hbm, x_bufs), (y_hbm, y_bufs),
                                 (mass_hbm, m_bufs)))
            for cp in cps:
                cp.start()
            return cps

        nvec = chunk // 16 + (1 if chunk % 16 else 0)
        pending = [None, None]
        pending[0] = dma_in(0, 0)

        for g in range(steps):
            slot = g % 2
            if g + 1 < steps:
                pending[1 - slot] = dma_in(g + 1, 1 - slot)
            for cp in pending[slot]:
                cp.wait()

            xbuf, ybuf, mbuf = x_bufs[slot], y_bufs[slot], m_bufs[slot]

            @plsc.parallel_loop(0, nvec, unroll=UNROLL)
            def _body(v, xbuf=xbuf, ybuf=ybuf, mbuf=mbuf):
                off = v * 16
                x = xbuf[pl.ds(off, 16)]
                y = ybuf[pl.ds(off, 16)]
                m = mbuf[pl.ds(off, 16)]
                s4 = (x * x + y * y) * 4.0
                j = jnp.minimum(s4, CLAMP).astype(jnp.int32)
                plsc.addupdate_scatter(hist, [j * 16 + lane], m)

        for b in range(R_BINS):
            acc = hist[pl.ds(b * b * 16, 16)]
            for j in range(b * b + 1, (b + 1) * (b + 1)):
                acc = acc + hist[pl.ds(j * 16, 16)]
            bins[pl.ds(b * 16, 16)] = acc

        pltpu.sync_copy(bins, shared.at[pl.ds(sid * R_BINS * 16, R_BINS * 16)])
        plsc.subcore_barrier()

        @pl.when(sid == 0)
        def _finish():
            pltpu.sync_copy(shared, stage)
            o0 = zero16
            o1 = zero16
            for b in range(R_BINS):
                acc = stage[pl.ds(b * 16, 16)]
                for t in range(1, 16):
                    acc = acc + stage[pl.ds(t * R_BINS * 16 + b * 16, 16)]
                tot = jnp.sum(acc) * (1.0 / (math.pi * 0.25 * (2 * b + 1)))
                if b < 16:
                    o0 = jnp.where(lane == b, tot, o0)
                else:
                    o1 = jnp.where(lane == (b - 16), tot, o1)
            outbuf[pl.ds(0, 16)] = o0
            outbuf[pl.ds(16, 16)] = o1
            pltpu.sync_copy(outbuf, out_hbm.at[pl.ds(cid * 32, 32)])

    return _sc_hist


def kernel(positions, masses):
    hists = []
    base = 0
    for n, chunk, steps in PARTS:
        hists.append(_make_sc_hist(n, chunk, steps)(
            positions[base:base + n, 0], positions[base:base + n, 1],
            masses[base:base + n]))
        base += n
    p = sum(hists[1:], hists[0]).reshape(2, 32)
    return p[0, :R_BINS] + p[1, :R_BINS]


# speedup vs baseline: 2.0747x; 2.0747x over previous
import functools
import math

import jax
import jax.numpy as jnp
from jax import lax
from jax.experimental import pallas as pl
from jax.experimental.pallas import tpu as pltpu
from jax.experimental.pallas import tpu_sc as plsc

N = 4_000_000
R_BINS = 20
NUM_TILES = 32
PARTS = ((3_200_000, 5000, 20), (800_000, 5000, 5))
VECS = 320
UNROLL = 4
BUF = VECS * 16
HIST_ROWS = R_BINS * R_BINS + 1
CLAMP = float(R_BINS * R_BINS)

_mesh = plsc.VectorSubcoreMesh(core_axis_name="c", subcore_axis_name="s")


@functools.lru_cache(maxsize=None)
def _make_sc_hist(n, chunk, steps, base):
    per_tile = n // NUM_TILES
    assert chunk * steps == per_tile and chunk % 8 == 0 and per_tile % 8 == 0
    assert chunk <= VECS * 16

    @functools.partial(
        pl.kernel,
        out_type=jax.ShapeDtypeStruct((64,), jnp.float32),
        mesh=_mesh,
        compiler_params=pltpu.CompilerParams(needs_layout_passes=False),
        scratch_types=[
            pltpu.VMEM((BUF,), jnp.float32),
            pltpu.VMEM((BUF,), jnp.float32),
            pltpu.VMEM((BUF,), jnp.float32),
            pltpu.VMEM((BUF,), jnp.float32),
            pltpu.VMEM((BUF,), jnp.float32),
            pltpu.VMEM((BUF,), jnp.float32),
            pltpu.VMEM((HIST_ROWS * 16,), jnp.float32),
            pltpu.VMEM((R_BINS * 16,), jnp.float32),
            pltpu.VMEM((16 * R_BINS * 16,), jnp.float32),
            pltpu.VMEM((32,), jnp.float32),
            pltpu.VMEM_SHARED((16 * R_BINS * 16,), jnp.float32),
            pltpu.SemaphoreType.DMA,
            pltpu.SemaphoreType.DMA,
        ],
    )
    def _sc_hist(x_hbm, y_hbm, mass_hbm, out_hbm, x_a, x_b, y_a, y_b, m_a, m_b,
                 hist, bins, stage, outbuf, shared, sem_a, sem_b):
        cid = lax.axis_index("c")
        sid = lax.axis_index("s")
        wid = sid * 2 + cid
        pbase = wid * per_tile

        zero16 = jnp.zeros((16,), jnp.float32)
        lane = lax.iota(jnp.int32, 16)

        x_bufs = (x_a, x_b)
        y_bufs = (y_a, y_b)
        m_bufs = (m_a, m_b)
        sems = (sem_a, sem_b)

        if chunk % 16:
            pad = ((chunk + 15) // 16) * 16 - 16
            for buf in (x_a, x_b, y_a, y_b, m_a, m_b):
                buf[pl.ds(pad, 16)] = zero16

        def _zh(i, c):
            hist[pl.ds(i * 16, 16)] = zero16
            return c
        lax.fori_loop(0, HIST_ROWS, _zh, 0)

        def dma_in(g, slot):
            p0 = pbase + g * chunk
            cps = (
                pltpu.make_async_copy(
                    x_hbm.at[pl.ds(p0, chunk)],
                    x_bufs[slot].at[pl.ds(0, chunk)], sems[slot]),
                pltpu.make_async_copy(
                    y_hbm.at[pl.ds(p0, chunk)],
                    y_bufs[slot].at[pl.ds(0, chunk)], sems[slot]),
                pltpu.make_async_copy(
                    mass_hbm.at[pl.ds(base + p0, chunk)],
                    m_bufs[slot].at[pl.ds(0, chunk)], sems[slot]),
            )
            for cp in cps:
                cp.start()
            return cps

        nvec = chunk // 16 + (1 if chunk % 16 else 0)
        pending = [None, None]
        pending[0] = dma_in(0, 0)

        for g in range(steps):
            slot = g % 2
            if g + 1 < steps:
                pending[1 - slot] = dma_in(g + 1, 1 - slot)
            for cp in pending[slot]:
                cp.wait()

            xbuf, ybuf, mbuf = x_bufs[slot], y_bufs[slot], m_bufs[slot]

            @plsc.parallel_loop(0, nvec, unroll=UNROLL)
            def _body(v, xbuf=xbuf, ybuf=ybuf, mbuf=mbuf):
                off = v * 16
                x = xbuf[pl.ds(off, 16)]
                y = ybuf[pl.ds(off, 16)]
                m = mbuf[pl.ds(off, 16)]
                s4 = (x * x + y * y) * 4.0
                j = jnp.minimum(s4, CLAMP).astype(jnp.int32)
                plsc.addupdate_scatter(hist, [j * 16 + lane], m)

        for b in range(R_BINS):
            acc = hist[pl.ds(b * b * 16, 16)]
            for j in range(b * b + 1, (b + 1) * (b + 1)):
                acc = acc + hist[pl.ds(j * 16, 16)]
            bins[pl.ds(b * 16, 16)] = acc

        pltpu.sync_copy(bins, shared.at[pl.ds(sid * R_BINS * 16, R_BINS * 16)])
        plsc.subcore_barrier()

        @pl.when(sid == 0)
        def _finish():
            pltpu.sync_copy(shared, stage)
            o0 = zero16
            o1 = zero16
            for b in range(R_BINS):
                acc = stage[pl.ds(b * 16, 16)]
                for t in range(1, 16):
                    acc = acc + stage[pl.ds(t * R_BINS * 16 + b * 16, 16)]
                tot = jnp.sum(acc) * (1.0 / (math.pi * 0.25 * (2 * b + 1)))
                if b < 16:
                    o0 = jnp.where(lane == b, tot, o0)
                else:
                    o1 = jnp.where(lane == (b - 16), tot, o1)
            outbuf[pl.ds(0, 16)] = o0
            outbuf[pl.ds(16, 16)] = o1
            pltpu.sync_copy(outbuf, out_hbm.at[pl.ds(cid * 32, 32)])

    return _sc_hist


def kernel(positions, masses):
    hists = []
    base = 0
    for n, chunk, steps in PARTS:
        hists.append(_make_sc_hist(n, chunk, steps, base)(
            positions[base:base + n, 0], positions[base:base + n, 1],
            masses))
        base += n
    p = sum(hists[1:], hists[0]).reshape(2, 32)
    return p[0, :R_BINS] + p[1, :R_BINS]
